# Initial kernel scaffold; baseline (speedup 1.0000x reference)
import jax
import jax.numpy as jnp
from jax.experimental import pallas as pl


def kernel(hidden_states, time_of_day, day_of_week, pos_emb, tod_W, tod_b, dow_emb, ln_gamma, ln_beta):
    raise NotImplementedError("write your pallas kernel here")


# speedup vs baseline: 3.8525x; 3.8525x over previous
import math

import jax
import jax.numpy as jnp
from jax.experimental import pallas as pl

_EPS = 1e-12
_TWO_PI = 2.0 * math.pi


def _fused_kernel(hid_ref, pos_ref, tod_ref, day_ref, wcat_ref, todb_ref,
                  gamma_ref, beta_ref, out_ref):
    x = hid_ref[0]
    p = pos_ref[...]
    tod = tod_ref[0, 0]
    day = day_ref[0, 0]

    rad = tod * _TWO_PI
    sin_t = jnp.sin(rad)
    cos_t = jnp.cos(rad)

    bs = x.shape[0]
    col = jax.lax.broadcasted_iota(jnp.int32, (bs, 16), 1)
    onehot = (col == day + 2).astype(jnp.float32)
    m = jnp.where(col == 0, sin_t, jnp.where(col == 1, cos_t, onehot))
    extra = jnp.dot(m, wcat_ref[...],
                    preferred_element_type=jnp.float32)

    h = x + p + extra + todb_ref[...]
    mean = jnp.mean(h, axis=1, keepdims=True)
    c = h - mean
    var = jnp.mean(c * c, axis=1, keepdims=True)
    normed = c * jax.lax.rsqrt(var + _EPS)
    out_ref[0] = normed * gamma_ref[...] + beta_ref[...]


def kernel(hidden_states, time_of_day, day_of_week, pos_emb, tod_W, tod_b,
           dow_emb, ln_gamma, ln_beta):
    B, S, H = hidden_states.shape
    BS = 512
    NSB = S // BS

    wcat = jnp.concatenate(
        [tod_W, dow_emb,
         jnp.zeros((16 - 2 - dow_emb.shape[0], H), jnp.float32)], axis=0)

    tod4 = time_of_day.reshape(B, NSB, BS, 1)
    day4 = day_of_week.astype(jnp.int32).reshape(B, NSB, BS, 1)
    todb2 = tod_b.reshape(1, H)
    gamma2 = ln_gamma.reshape(1, H)
    beta2 = ln_beta.reshape(1, H)

    grid = (NSB, B)
    out = pl.pallas_call(
        _fused_kernel,
        grid=grid,
        in_specs=[
            pl.BlockSpec((1, BS, H), lambda s, b: (b, s, 0)),
            pl.BlockSpec((BS, H), lambda s, b: (s, 0)),
            pl.BlockSpec((1, 1, BS, 1), lambda s, b: (b, s, 0, 0)),
            pl.BlockSpec((1, 1, BS, 1), lambda s, b: (b, s, 0, 0)),
            pl.BlockSpec((16, H), lambda s, b: (0, 0)),
            pl.BlockSpec((1, H), lambda s, b: (0, 0)),
            pl.BlockSpec((1, H), lambda s, b: (0, 0)),
            pl.BlockSpec((1, H), lambda s, b: (0, 0)),
        ],
        out_specs=pl.BlockSpec((1, BS, H), lambda s, b: (b, s, 0)),
        out_shape=jax.ShapeDtypeStruct((B, S, H), jnp.float32),
    )(hidden_states, pos_emb, tod4, day4, wcat, todb2, gamma2, beta2)
    return out
